# baseline (speedup 1.0000x reference)
import functools

import jax
import jax.numpy as jnp
import numpy as np
from jax import lax
from jax.experimental import pallas as pl
from jax.experimental.pallas import tpu as pltpu
from jax.experimental.pallas import tpu_sc as plsc

L = 16
KM = 4
KC = 4

SC_COLS = 256
TC_DT = 512
TC_ST = 256


def _sc_spatial_conv(Yf, xP2, W, d_cols, d_base):
  BF, N, _ = Yf.shape
  C = W.shape[0]
  BCF = BF * C
  DB = 128
  SPW = N // 16
  SB = 16
  NCHUNK = SPW // SB
  NBI = 32

  mesh = plsc.VectorSubcoreMesh(core_axis_name="cc", subcore_axis_name="ss")

  @functools.partial(
      pl.kernel,
      out_type=jax.ShapeDtypeStruct((BCF, d_cols), jnp.float32),
      mesh=mesh,
      scratch_types=[
          pltpu.VMEM((2, BF, SB, DB), jnp.float32),
          pltpu.VMEM((C, SB, DB), jnp.float32),
          pltpu.VMEM((SB, 512), jnp.float32),
          pltpu.VMEM((BCF, DB), jnp.float32),
          pltpu.VMEM((8, DB), jnp.float32),
          pltpu.VMEM((8, DB), jnp.float32),
          pltpu.VMEM_SHARED((16, BCF, DB), jnp.float32),
          pltpu.SemaphoreType.DMA((2,)),
      ],
  )
  def sck(Y_hbm, xP_hbm, W_hbm, out_hbm, Yb, Wb, xb, accb, rbuf, tbuf,
          shared, sem):
    cc = lax.axis_index("cc")
    ss = lax.axis_index("ss")
    d0 = d_base + cc * DB
    s_base = ss * SPW

    zero = jnp.zeros((L,), jnp.float32)

    def zrow(r, carry):
      for k in range(DB // L):
        accb[r, pl.ds(k * L, L)] = zero
      return carry

    lax.fori_loop(0, BCF, zrow, 0)

    def issue(ci, slot):
      s0 = s_base + ci * SB
      for g in range(BF):
        pltpu.async_copy(
            Y_hbm.at[g, pl.ds(s0, SB), pl.ds(d0, DB)],
            Yb.at[slot, g], sem.at[slot])

    def drain(slot):
      for g in range(BF):
        pltpu.make_async_copy(
            Y_hbm.at[0, pl.ds(0, SB), pl.ds(0, DB)],
            Yb.at[slot, 0], sem.at[slot]).wait()

    issue(0, 0)

    def chunk_body(ci, carry):
      cur = lax.rem(ci, 2)

      @pl.when(ci + 1 < NCHUNK)
      def _():
        issue(ci + 1, 1 - cur)

      drain(cur)
      s0c = s_base + ci * SB
      for c in range(C):
        pltpu.sync_copy(W_hbm.at[c, pl.ds(s0c, SB), pl.ds(d0, DB)], Wb.at[c])
      pltpu.sync_copy(xP_hbm.at[pl.ds(s0c, SB)], xb)

      def bi_body(bi, carry2):
        g0 = lax.rem(bi, 8) * 2
        c0 = (bi // 8) * 2
        for dh in range(2):
          init = []
          for j in range(4):
            for k in range(4):
              init.append(accb[bi * 4 + j, pl.ds(dh * 64 + k * L, L)])

          def sbody(s, accs, g0=g0, c0=c0, dh=dh, cur=cur, bi=bi):
            xv = xb[s, pl.ds(bi * L, L)]
            y = [[Yb[cur, g0 + gi, s, pl.ds(dh * 64 + k * L, L)]
                  for k in range(4)] for gi in range(2)]
            w = [[Wb[c0 + ci_, s, pl.ds(dh * 64 + k * L, L)]
                  for k in range(4)] for ci_ in range(2)]
            new = []
            for j, (gi, ci_) in enumerate(((0, 0), (0, 1), (1, 0), (1, 1))):
              xsv = jnp.full((L,), xv[j], dtype=jnp.float32)
              for k in range(4):
                new.append(accs[j * 4 + k] + xsv * (y[gi][k] * w[ci_][k]))
            return tuple(new)

          final = lax.fori_loop(0, SB, sbody, tuple(init))
          idx = 0
          for j in range(4):
            for k in range(4):
              accb[bi * 4 + j, pl.ds(dh * 64 + k * L, L)] = final[idx]
              idx += 1
        return carry2

      lax.fori_loop(0, NBI, bi_body, 0)
      return carry

    lax.fori_loop(0, NCHUNK, chunk_body, 0)

    pltpu.sync_copy(accb, shared.at[ss])
    plsc.subcore_barrier()

    base = ss * 8
    pltpu.sync_copy(shared.at[0, pl.ds(base, 8)], rbuf)

    def red_body(t, carry):
      pltpu.sync_copy(shared.at[t, pl.ds(base, 8)], tbuf)
      for rr in range(8):
        for k in range(DB // L):
          rbuf[rr, pl.ds(k * L, L)] = (rbuf[rr, pl.ds(k * L, L)] +
                                       tbuf[rr, pl.ds(k * L, L)])
      return carry

    lax.fori_loop(1, 16, red_body, 0)

    for rr in range(8):
      for k in range(DB // L):
        rbuf[rr, pl.ds(k * L, L)] = jnp.maximum(rbuf[rr, pl.ds(k * L, L)],
                                                0.0)
    pltpu.sync_copy(rbuf, out_hbm.at[pl.ds(base, 8), pl.ds(cc * DB, DB)])

  return sck(Yf, xP2, W)


def _tc_spatial_conv(Yf, xT2, W, dT, sT, d_lo, d_cols):
  BF, N, _ = Yf.shape
  C = W.shape[0]
  BCF = xT2.shape[1]
  n_s = N // sT
  grid = (d_cols // dT, n_s)

  def body(y_ref, w_ref, x_ref, o_ref):
    j = pl.program_id(1)
    rows = [None] * BCF
    for g in range(BF):
      yg = y_ref[g]
      for c in range(C):
        r = (g // 8) * (C * 8) + c * 8 + (g % 8)
        xcol = x_ref[:, r]
        bcx = jnp.broadcast_to(xcol[:, None], (sT, dT))
        u = (yg * bcx) * w_ref[c]
        rows[r] = jnp.sum(u, axis=0, keepdims=True)
    res = jnp.concatenate(rows, axis=0)

    @pl.when(j == 0)
    def _():
      o_ref[...] = res

    @pl.when(j > 0)
    def _():
      o_ref[...] += res

    @pl.when(j == n_s - 1)
    def _():
      o_ref[...] = jnp.maximum(o_ref[...], 0.0)

  return pl.pallas_call(
      body,
      grid=grid,
      in_specs=[
          pl.BlockSpec((BF, sT, dT), lambda i, j: (0, j, i + d_lo // dT)),
          pl.BlockSpec((C, sT, dT), lambda i, j: (0, j, i + d_lo // dT)),
          pl.BlockSpec((sT, BCF), lambda i, j: (j, 0)),
      ],
      out_specs=pl.BlockSpec((BCF, dT), lambda i, j: (0, i)),
      out_shape=jax.ShapeDtypeStruct((BCF, d_cols), jnp.float32),
  )(Yf, W, xT2)


@jax.jit
def kernel(Y, x, W):
  B, F, N, _ = Y.shape
  C = x.shape[1]
  BCF = B * C * F
  BF = B * F
  Yf = Y.reshape(BF, N, N)
  xf = x.reshape(BCF, N)

  xT2 = xf.T

  tc_cols = N - SC_COLS
  out_tc = _tc_spatial_conv(Yf, xT2, W, tc_cols, TC_ST, 0, tc_cols)

  if SC_COLS:
    sel = np.zeros((32, L), dtype=np.int32)
    pos = np.empty(BCF, dtype=np.int32)
    for bi in range(32):
      mb, cb = bi % 8, bi // 8
      for j in range(4):
        gi, ci = j // 2, j % 2
        g = mb * 2 + gi
        c = cb * 2 + ci
        b_, f_ = g // F, g % F
        r = b_ * C * F + c * F + f_
        sel[bi, j] = r
        pos[r] = bi * 4 + j
    xP2 = xf[sel.reshape(-1), :].T.reshape(N, 32 * L)
    raw = _sc_spatial_conv(Yf, xP2, W, SC_COLS, tc_cols)
    out = jnp.concatenate([out_tc, raw[pos]], axis=1)
  else:
    out = out_tc
  return out.reshape(B, C, F, N)


# speedup vs baseline: 1.3078x; 1.3078x over previous
import functools

import jax
import jax.numpy as jnp
import numpy as np
from jax import lax
from jax.experimental import pallas as pl
from jax.experimental.pallas import tpu as pltpu
from jax.experimental.pallas import tpu_sc as plsc

L = 16
KM = 4
KC = 4

SC_COLS = 256
TC_DT = 512
TC_ST = 256


def _sc_spatial_conv(Yf, xP2, W, d_cols, d_base):
  BF, N, _ = Yf.shape
  C = W.shape[0]
  BCF = BF * C
  DB = 128
  SPW = N // 16
  SB = 8
  NCHUNK = SPW // SB
  NBI = 32

  mesh = plsc.VectorSubcoreMesh(core_axis_name="cc", subcore_axis_name="ss")

  @functools.partial(
      pl.kernel,
      out_type=jax.ShapeDtypeStruct((BCF, d_cols), jnp.float32),
      mesh=mesh,
      scratch_types=[
          pltpu.VMEM((2, BF, SB, DB), jnp.float32),
          pltpu.VMEM((2, C, SB, DB), jnp.float32),
          pltpu.VMEM((2, SB, 512), jnp.float32),
          pltpu.VMEM((BCF, DB), jnp.float32),
          pltpu.VMEM((8, DB), jnp.float32),
          pltpu.VMEM((8, DB), jnp.float32),
          pltpu.VMEM_SHARED((16, BCF, DB), jnp.float32),
          pltpu.SemaphoreType.DMA((2,)),
      ],
  )
  def sck(Y_hbm, xP_hbm, W_hbm, out_hbm, Yb, Wb, xb, accb, rbuf, tbuf,
          shared, sem):
    cc = lax.axis_index("cc")
    ss = lax.axis_index("ss")
    d0 = d_base + cc * DB
    s_base = ss * SPW

    zero = jnp.zeros((L,), jnp.float32)

    def zrow(r, carry):
      for k in range(DB // L):
        accb[r, pl.ds(k * L, L)] = zero
      return carry

    lax.fori_loop(0, BCF, zrow, 0)

    def issue(ci, slot):
      s0 = s_base + ci * SB
      for g in range(BF):
        pltpu.async_copy(
            Y_hbm.at[g, pl.ds(s0, SB), pl.ds(d0, DB)],
            Yb.at[slot, g], sem.at[slot])
      for c in range(C):
        pltpu.async_copy(
            W_hbm.at[c, pl.ds(s0, SB), pl.ds(d0, DB)],
            Wb.at[slot, c], sem.at[slot])
      pltpu.async_copy(xP_hbm.at[pl.ds(s0, SB)], xb.at[slot], sem.at[slot])

    def drain(slot):
      for g in range(BF):
        pltpu.make_async_copy(
            Y_hbm.at[0, pl.ds(0, SB), pl.ds(0, DB)],
            Yb.at[slot, 0], sem.at[slot]).wait()
      for c in range(C):
        pltpu.make_async_copy(
            W_hbm.at[0, pl.ds(0, SB), pl.ds(0, DB)],
            Wb.at[slot, 0], sem.at[slot]).wait()
      pltpu.make_async_copy(xP_hbm.at[pl.ds(0, SB)], xb.at[slot],
                            sem.at[slot]).wait()

    issue(0, 0)

    def chunk_body(ci, carry):
      cur = lax.rem(ci, 2)

      @pl.when(ci + 1 < NCHUNK)
      def _():
        issue(ci + 1, 1 - cur)

      drain(cur)

      def bi_body(bi, carry2):
        g0 = lax.rem(bi, 8) * 2
        c0 = (bi // 8) * 2
        for dh in range(2):
          init = []
          for j in range(4):
            for k in range(4):
              init.append(accb[bi * 4 + j, pl.ds(dh * 64 + k * L, L)])

          def sbody(s, accs, g0=g0, c0=c0, dh=dh, cur=cur, bi=bi):
            xv = xb[cur, s, pl.ds(bi * L, L)]
            y = [[Yb[cur, g0 + gi, s, pl.ds(dh * 64 + k * L, L)]
                  for k in range(4)] for gi in range(2)]
            w = [[Wb[cur, c0 + ci_, s, pl.ds(dh * 64 + k * L, L)]
                  for k in range(4)] for ci_ in range(2)]
            new = []
            for j, (gi, ci_) in enumerate(((0, 0), (0, 1), (1, 0), (1, 1))):
              xsv = jnp.full((L,), xv[j], dtype=jnp.float32)
              for k in range(4):
                new.append(accs[j * 4 + k] + xsv * (y[gi][k] * w[ci_][k]))
            return tuple(new)

          final = lax.fori_loop(0, SB, sbody, tuple(init))
          idx = 0
          for j in range(4):
            for k in range(4):
              accb[bi * 4 + j, pl.ds(dh * 64 + k * L, L)] = final[idx]
              idx += 1
        return carry2

      lax.fori_loop(0, NBI, bi_body, 0)
      return carry

    lax.fori_loop(0, NCHUNK, chunk_body, 0)

    pltpu.sync_copy(accb, shared.at[ss])
    plsc.subcore_barrier()

    base = ss * 8
    pltpu.sync_copy(shared.at[0, pl.ds(base, 8)], rbuf)

    def red_body(t, carry):
      pltpu.sync_copy(shared.at[t, pl.ds(base, 8)], tbuf)
      for rr in range(8):
        for k in range(DB // L):
          rbuf[rr, pl.ds(k * L, L)] = (rbuf[rr, pl.ds(k * L, L)] +
                                       tbuf[rr, pl.ds(k * L, L)])
      return carry

    lax.fori_loop(1, 16, red_body, 0)

    for rr in range(8):
      for k in range(DB // L):
        rbuf[rr, pl.ds(k * L, L)] = jnp.maximum(rbuf[rr, pl.ds(k * L, L)],
                                                0.0)
    pltpu.sync_copy(rbuf, out_hbm.at[pl.ds(base, 8), pl.ds(cc * DB, DB)])

  return sck(Yf, xP2, W)


def _tc_spatial_conv(Yf, xT2, W, dT, sT, d_lo, d_cols):
  BF, N, _ = Yf.shape
  C = W.shape[0]
  BCF = xT2.shape[1]
  n_s = N // sT
  grid = (d_cols // dT, n_s)

  def body(y_ref, w_ref, x_ref, o_ref):
    j = pl.program_id(1)
    rows = [None] * BCF
    for g in range(BF):
      yg = y_ref[g]
      for c in range(C):
        r = (g // 8) * (C * 8) + c * 8 + (g % 8)
        xcol = x_ref[:, r]
        bcx = jnp.broadcast_to(xcol[:, None], (sT, dT))
        u = (yg * bcx) * w_ref[c]
        rows[r] = jnp.sum(u, axis=0, keepdims=True)
    res = jnp.concatenate(rows, axis=0)

    @pl.when(j == 0)
    def _():
      o_ref[...] = res

    @pl.when(j > 0)
    def _():
      o_ref[...] += res

    @pl.when(j == n_s - 1)
    def _():
      o_ref[...] = jnp.maximum(o_ref[...], 0.0)

  return pl.pallas_call(
      body,
      grid=grid,
      in_specs=[
          pl.BlockSpec((BF, sT, dT), lambda i, j: (0, j, i + d_lo // dT)),
          pl.BlockSpec((C, sT, dT), lambda i, j: (0, j, i + d_lo // dT)),
          pl.BlockSpec((sT, BCF), lambda i, j: (j, 0)),
      ],
      out_specs=pl.BlockSpec((BCF, dT), lambda i, j: (0, i)),
      out_shape=jax.ShapeDtypeStruct((BCF, d_cols), jnp.float32),
  )(Yf, W, xT2)


@jax.jit
def kernel(Y, x, W):
  B, F, N, _ = Y.shape
  C = x.shape[1]
  BCF = B * C * F
  BF = B * F
  Yf = Y.reshape(BF, N, N)
  xf = x.reshape(BCF, N)

  xT2 = xf.T

  tc_cols = N - SC_COLS
  out_tc = _tc_spatial_conv(Yf, xT2, W, tc_cols, TC_ST, 0, tc_cols)

  if SC_COLS:
    sel = np.zeros((32, L), dtype=np.int32)
    pos = np.empty(BCF, dtype=np.int32)
    for bi in range(32):
      mb, cb = bi % 8, bi // 8
      for j in range(4):
        gi, ci = j // 2, j % 2
        g = mb * 2 + gi
        c = cb * 2 + ci
        b_, f_ = g // F, g % F
        r = b_ * C * F + c * F + f_
        sel[bi, j] = r
        pos[r] = bi * 4 + j
    xP2 = xf[sel.reshape(-1), :].T.reshape(N, 32 * L)
    raw = _sc_spatial_conv(Yf, xP2, W, SC_COLS, tc_cols)
    out = jnp.concatenate([out_tc, raw[pos]], axis=1)
  else:
    out = out_tc
  return out.reshape(B, C, F, N)


# speedup vs baseline: 1.3632x; 1.0424x over previous
import functools

import jax
import jax.numpy as jnp
import numpy as np
from jax import lax
from jax.experimental import pallas as pl
from jax.experimental.pallas import tpu as pltpu
from jax.experimental.pallas import tpu_sc as plsc

L = 16
KM = 4
KC = 4

SC_COLS = 256
TC_DT = 512
TC_ST = 256


def _sc_spatial_conv(Yf, xP2, W, d_cols, d_base):
  BF, N, _ = Yf.shape
  C = W.shape[0]
  BCF = BF * C
  DB = 128
  SPW = N // 16
  SB = 8
  NCHUNK = SPW // SB
  NBI = 32

  mesh = plsc.VectorSubcoreMesh(core_axis_name="cc", subcore_axis_name="ss")

  @functools.partial(
      pl.kernel,
      out_type=jax.ShapeDtypeStruct((BCF, d_cols), jnp.float32),
      mesh=mesh,
      scratch_types=[
          pltpu.VMEM((2, BF, SB, DB), jnp.float32),
          pltpu.VMEM((2, C, SB, DB), jnp.float32),
          pltpu.VMEM((2, SB, 512), jnp.float32),
          pltpu.VMEM((BCF, DB), jnp.float32),
          pltpu.VMEM((8, DB), jnp.float32),
          pltpu.VMEM((8, DB), jnp.float32),
          pltpu.VMEM_SHARED((16, BCF, DB), jnp.float32),
          pltpu.SemaphoreType.DMA((2,)),
      ],
  )
  def sck(Y_hbm, xP_hbm, W_hbm, out_hbm, Yb, Wb, xb, accb, rbuf, tbuf,
          shared, sem):
    cc = lax.axis_index("cc")
    ss = lax.axis_index("ss")
    d0 = d_base + cc * DB
    s_base = ss * SPW

    zero = jnp.zeros((L,), jnp.float32)

    def zrow(r, carry):
      for k in range(DB // L):
        accb[r, pl.ds(k * L, L)] = zero
      return carry

    lax.fori_loop(0, BCF, zrow, 0)

    def issue(ci, slot):
      s0 = s_base + ci * SB
      for g in range(BF):
        pltpu.async_copy(
            Y_hbm.at[g, pl.ds(s0, SB), pl.ds(d0, DB)],
            Yb.at[slot, g], sem.at[slot])
      for c in range(C):
        pltpu.async_copy(
            W_hbm.at[c, pl.ds(s0, SB), pl.ds(d0, DB)],
            Wb.at[slot, c], sem.at[slot])
      pltpu.async_copy(xP_hbm.at[pl.ds(s0, SB)], xb.at[slot], sem.at[slot])

    def drain(slot):
      for g in range(BF):
        pltpu.make_async_copy(
            Y_hbm.at[0, pl.ds(0, SB), pl.ds(0, DB)],
            Yb.at[slot, 0], sem.at[slot]).wait()
      for c in range(C):
        pltpu.make_async_copy(
            W_hbm.at[0, pl.ds(0, SB), pl.ds(0, DB)],
            Wb.at[slot, 0], sem.at[slot]).wait()
      pltpu.make_async_copy(xP_hbm.at[pl.ds(0, SB)], xb.at[slot],
                            sem.at[slot]).wait()

    issue(0, 0)

    def chunk_body(ci, carry):
      cur = lax.rem(ci, 2)

      @pl.when(ci + 1 < NCHUNK)
      def _():
        issue(ci + 1, 1 - cur)

      drain(cur)

      def bi_body(bi, carry2):
        g0 = lax.rem(bi, 8) * 2
        c0 = (bi // 8) * 2
        for dh in range(2):
          init = []
          for j in range(4):
            for k in range(4):
              init.append(accb[bi * 4 + j, pl.ds(dh * 64 + k * L, L)])

          def sbody(s, accs, g0=g0, c0=c0, dh=dh, cur=cur, bi=bi):
            xv = xb[cur, s, pl.ds(bi * L, L)]
            y = [[Yb[cur, g0 + gi, s, pl.ds(dh * 64 + k * L, L)]
                  for k in range(4)] for gi in range(2)]
            w = [[Wb[cur, c0 + ci_, s, pl.ds(dh * 64 + k * L, L)]
                  for k in range(4)] for ci_ in range(2)]
            new = []
            for j, (gi, ci_) in enumerate(((0, 0), (0, 1), (1, 0), (1, 1))):
              xsv = jnp.full((L,), xv[j], dtype=jnp.float32)
              for k in range(4):
                new.append(accs[j * 4 + k] + xsv * (y[gi][k] * w[ci_][k]))
            return tuple(new)

          final = lax.fori_loop(0, SB, sbody, tuple(init))
          idx = 0
          for j in range(4):
            for k in range(4):
              accb[bi * 4 + j, pl.ds(dh * 64 + k * L, L)] = final[idx]
              idx += 1
        return carry2

      lax.fori_loop(0, NBI, bi_body, 0)
      return carry

    lax.fori_loop(0, NCHUNK, chunk_body, 0)

    pltpu.sync_copy(accb, shared.at[ss])
    plsc.subcore_barrier()

    base = ss * 8
    pltpu.sync_copy(shared.at[0, pl.ds(base, 8)], rbuf)

    def red_body(t, carry):
      pltpu.sync_copy(shared.at[t, pl.ds(base, 8)], tbuf)
      for rr in range(8):
        for k in range(DB // L):
          rbuf[rr, pl.ds(k * L, L)] = (rbuf[rr, pl.ds(k * L, L)] +
                                       tbuf[rr, pl.ds(k * L, L)])
      return carry

    lax.fori_loop(1, 16, red_body, 0)

    for rr in range(8):
      for k in range(DB // L):
        rbuf[rr, pl.ds(k * L, L)] = jnp.maximum(rbuf[rr, pl.ds(k * L, L)],
                                                0.0)
    pltpu.sync_copy(rbuf, out_hbm.at[pl.ds(base, 8), pl.ds(cc * DB, DB)])

  return sck(Yf, xP2, W)


def _tc_spatial_conv(Yf, xT2, W, dT, sT, d_lo, d_cols):
  BF, N, _ = Yf.shape
  C = W.shape[0]
  BCF = xT2.shape[1]
  n_s = N // sT
  grid = (d_cols // dT, n_s)

  def body(y_ref, w_ref, x_ref, o_ref):
    j = pl.program_id(1)
    rows = []
    for g in range(BF):
      yg = y_ref[g]
      for c in range(C):
        xcol = x_ref[:, g * C + c]
        bcx = jnp.broadcast_to(xcol[:, None], (sT, dT))
        u = (yg * bcx) * w_ref[c]
        rows.append(jnp.sum(u, axis=0, keepdims=True))
    res = jnp.concatenate(rows, axis=0)

    @pl.when(j == 0)
    def _():
      o_ref[...] = res

    @pl.when(j > 0)
    def _():
      o_ref[...] += res

    @pl.when(j == n_s - 1)
    def _():
      o_ref[...] = jnp.maximum(o_ref[...], 0.0)

  return pl.pallas_call(
      body,
      grid=grid,
      in_specs=[
          pl.BlockSpec((BF, sT, dT), lambda i, j: (0, j, i + d_lo // dT)),
          pl.BlockSpec((C, sT, dT), lambda i, j: (0, j, i + d_lo // dT)),
          pl.BlockSpec((sT, BCF), lambda i, j: (j, 0)),
      ],
      out_specs=pl.BlockSpec((BCF, dT), lambda i, j: (0, i)),
      out_shape=jax.ShapeDtypeStruct((BCF, d_cols), jnp.float32),
  )(Yf, W, xT2)


@jax.jit
def kernel(Y, x, W):
  B, F, N, _ = Y.shape
  C = x.shape[1]
  BCF = B * C * F
  BF = B * F
  Yf = Y.reshape(BF, N, N)
  xf = x.reshape(BCF, N)

  rmap = np.empty(BCF, dtype=np.int32)
  for g in range(BF):
    b_, f_ = g // F, g % F
    for c in range(C):
      rmap[g * C + c] = b_ * C * F + c * F + f_
  inv = np.empty(BCF, dtype=np.int32)
  inv[rmap] = np.arange(BCF, dtype=np.int32)
  xT2 = xf[rmap, :].T

  tc_cols = N - SC_COLS
  out_tc = _tc_spatial_conv(Yf, xT2, W, tc_cols, TC_ST, 0, tc_cols)[inv]

  if SC_COLS:
    sel = np.zeros((32, L), dtype=np.int32)
    pos = np.empty(BCF, dtype=np.int32)
    for bi in range(32):
      mb, cb = bi % 8, bi // 8
      for j in range(4):
        gi, ci = j // 2, j % 2
        g = mb * 2 + gi
        c = cb * 2 + ci
        b_, f_ = g // F, g % F
        r = b_ * C * F + c * F + f_
        sel[bi, j] = r
        pos[r] = bi * 4 + j
    xP2 = xf[sel.reshape(-1), :].T.reshape(N, 32 * L)
    raw = _sc_spatial_conv(Yf, xP2, W, SC_COLS, tc_cols)
    out = jnp.concatenate([out_tc, raw[pos]], axis=1)
  else:
    out = out_tc
  return out.reshape(B, C, F, N)
